# baseline (speedup 1.0000x reference)
import jax
import jax.numpy as jnp
from jax import lax
from jax.experimental import pallas as pl
from jax.experimental.pallas import tpu as pltpu
from jax.experimental.pallas import tpu_sc as plsc

_ROWS = 16384
_COLS = 4096
_BLOCK_R = 512


def _copy_body(x_ref, o_ref):
    o_ref[...] = x_ref[...]


def _tc_copy(x):
    return pl.pallas_call(
        _copy_body,
        grid=(_ROWS // _BLOCK_R,),
        in_specs=[pl.BlockSpec((_BLOCK_R, _COLS), lambda i: (i, 0))],
        out_specs=pl.BlockSpec((_BLOCK_R, _COLS), lambda i: (i, 0)),
        out_shape=jax.ShapeDtypeStruct((_ROWS, _COLS), jnp.float32),
        compiler_params=pltpu.CompilerParams(
            dimension_semantics=("arbitrary",),
        ),
    )(x)


def _sc_row_body(i1_hbm, x_hbm, row_out_hbm, idx_spmem, idx_smem, row_spmem):
    cid = lax.axis_index("c")

    @pl.when(cid == 0)
    def _():
        pltpu.sync_copy(i1_hbm, idx_spmem)
        pltpu.sync_copy(idx_spmem, idx_smem)
        row = idx_smem[0]
        pltpu.sync_copy(x_hbm.at[pl.ds(row, 1), :], row_spmem)
        pltpu.sync_copy(row_spmem, row_out_hbm)


_sc_make_row = pl.kernel(
    _sc_row_body,
    out_type=jax.ShapeDtypeStruct((1, _COLS), jnp.float32),
    mesh=plsc.ScalarSubcoreMesh(axis_name="c", num_cores=1),
    compiler_params=pltpu.CompilerParams(needs_layout_passes=False),
    scratch_types=[
        pltpu.MemorySpace.VMEM_SHARED((1,), jnp.int32),
        pltpu.SMEM((1,), jnp.int32),
        pltpu.MemorySpace.VMEM_SHARED((1, _COLS), jnp.float32),
    ],
)


def _row_write_body(i1_ref, i2_ref, v_ref, copied_ref, row_ref, o_ref,
                    patched_ref, sem):
    row = i1_ref[0]
    col = i2_ref[0]
    lane = jax.lax.broadcasted_iota(jnp.int32, (1, _COLS), 1)
    patched_ref[...] = jnp.where(lane == col, v_ref[0], row_ref[...])
    put = pltpu.make_async_copy(patched_ref, o_ref.at[pl.ds(row, 1), :], sem)
    put.start()
    put.wait()


def _tc_row_write(copied, row, i1, i2, v):
    return pl.pallas_call(
        _row_write_body,
        in_specs=[
            pl.BlockSpec(memory_space=pltpu.SMEM),
            pl.BlockSpec(memory_space=pltpu.SMEM),
            pl.BlockSpec(memory_space=pltpu.SMEM),
            pl.BlockSpec(memory_space=pl.ANY),
            pl.BlockSpec(memory_space=pltpu.VMEM),
        ],
        out_specs=pl.BlockSpec(memory_space=pl.ANY),
        out_shape=jax.ShapeDtypeStruct((_ROWS, _COLS), jnp.float32),
        input_output_aliases={3: 0},
        scratch_shapes=[
            pltpu.VMEM((1, _COLS), jnp.float32),
            pltpu.SemaphoreType.DMA,
        ],
    )(i1, i2, v, copied, row)


def kernel(input, index1, index2, value):
    i1 = index1.astype(jnp.int32)
    i2 = index2.astype(jnp.int32)
    v = value.astype(jnp.float32)

    row = _sc_make_row(i1, input)
    copied = _tc_copy(input)
    return _tc_row_write(copied, row, i1, i2, v)


# speedup vs baseline: 1.0977x; 1.0977x over previous
import jax
import jax.numpy as jnp
from jax.experimental import pallas as pl
from jax.experimental.pallas import tpu as pltpu

_ROWS = 16384
_COLS = 4096
_BLOCK_R = 768


def _body(i1_ref, i2_ref, v_ref, x_ref, o_ref):
    i = pl.program_id(0)
    o_ref[...] = x_ref[...]
    row = i1_ref[0]
    col = i2_ref[0]
    blk_start = i * _BLOCK_R

    @pl.when((row >= blk_start) & (row < blk_start + _BLOCK_R))
    def _():
        r = row - blk_start
        row_vals = x_ref[pl.ds(r, 1), :]
        lane = jax.lax.broadcasted_iota(jnp.int32, (1, _COLS), 1)
        o_ref[pl.ds(r, 1), :] = jnp.where(lane == col, v_ref[0], row_vals)


def kernel(input, index1, index2, value):
    i1 = index1.astype(jnp.int32)
    i2 = index2.astype(jnp.int32)
    v = value.astype(jnp.float32)
    return pl.pallas_call(
        _body,
        grid=(pl.cdiv(_ROWS, _BLOCK_R),),
        in_specs=[
            pl.BlockSpec(memory_space=pltpu.SMEM),
            pl.BlockSpec(memory_space=pltpu.SMEM),
            pl.BlockSpec(memory_space=pltpu.SMEM),
            pl.BlockSpec((_BLOCK_R, _COLS), lambda i: (i, 0)),
        ],
        out_specs=pl.BlockSpec((_BLOCK_R, _COLS), lambda i: (i, 0)),
        out_shape=jax.ShapeDtypeStruct((_ROWS, _COLS), jnp.float32),
        compiler_params=pltpu.CompilerParams(
            dimension_semantics=("arbitrary",),
            vmem_limit_bytes=130 * 1024 * 1024,
        ),
    )(i1, i2, v, input)


# speedup vs baseline: 1.0980x; 1.0003x over previous
import jax
import jax.numpy as jnp
from jax.experimental import pallas as pl
from jax.experimental.pallas import tpu as pltpu

_ROWS = 16384
_COLS = 4096
_BLOCK_R = 960


def _body(i1_ref, i2_ref, v_ref, x_ref, o_ref):
    i = pl.program_id(0)
    o_ref[...] = x_ref[...]
    row = i1_ref[0]
    col = i2_ref[0]
    blk_start = i * _BLOCK_R

    @pl.when((row >= blk_start) & (row < blk_start + _BLOCK_R))
    def _():
        r = row - blk_start
        row_vals = x_ref[pl.ds(r, 1), :]
        lane = jax.lax.broadcasted_iota(jnp.int32, (1, _COLS), 1)
        o_ref[pl.ds(r, 1), :] = jnp.where(lane == col, v_ref[0], row_vals)


def kernel(input, index1, index2, value):
    i1 = index1.astype(jnp.int32)
    i2 = index2.astype(jnp.int32)
    v = value.astype(jnp.float32)
    return pl.pallas_call(
        _body,
        grid=(pl.cdiv(_ROWS, _BLOCK_R),),
        in_specs=[
            pl.BlockSpec(memory_space=pltpu.SMEM),
            pl.BlockSpec(memory_space=pltpu.SMEM),
            pl.BlockSpec(memory_space=pltpu.SMEM),
            pl.BlockSpec((_BLOCK_R, _COLS), lambda i: (i, 0)),
        ],
        out_specs=pl.BlockSpec((_BLOCK_R, _COLS), lambda i: (i, 0)),
        out_shape=jax.ShapeDtypeStruct((_ROWS, _COLS), jnp.float32),
        compiler_params=pltpu.CompilerParams(
            dimension_semantics=("arbitrary",),
            vmem_limit_bytes=130 * 1024 * 1024,
        ),
    )(i1, i2, v, input)
